# baseline (speedup 1.0000x reference)
import functools

import jax
import jax.numpy as jnp
from jax import lax
from jax.experimental import pallas as pl
from jax.experimental.pallas import tpu as pltpu
from jax.experimental.pallas import tpu_sc as plsc

EMBED = 16
PACKED = EMBED // 2
NUM_CORES = 2
NUM_SUBCORES = 16
NUM_WORKERS = NUM_CORES * NUM_SUBCORES
CHUNK = 128
NBUF = 5
GROUP = 8
MM_BLOCK = 512


def _pack_bf16_table(table):
    xi = lax.bitcast_convert_type(table, jnp.int32)
    rounded = xi + 0x7FFF + ((xi >> 16) & 1)
    bf = lax.shift_right_logical(rounded, 16)
    bf = bf.reshape(table.shape[0], PACKED, 2)
    packed = bf[:, :, 0] | (bf[:, :, 1] << 16)
    return lax.bitcast_convert_type(packed, jnp.float32)


def _sc_gather(table, idx_flat):
    num_indices = idx_flat.shape[0]
    per_worker = num_indices // NUM_WORKERS
    num_chunks = per_worker // CHUNK
    assert per_worker % CHUNK == 0 and num_chunks % NBUF == 0

    mesh = plsc.VectorSubcoreMesh(core_axis_name="c", subcore_axis_name="s")

    scratch = [pltpu.VMEM((per_worker,), jnp.int32)]
    scratch += [pltpu.VMEM((CHUNK, PACKED), table.dtype) for _ in range(NBUF)]
    scratch += [pltpu.SemaphoreType.DMA for _ in range(NBUF)]

    @functools.partial(
        pl.kernel,
        mesh=mesh,
        out_type=jax.ShapeDtypeStruct((num_indices, PACKED), table.dtype),
        scratch_types=scratch,
        compiler_params=pltpu.CompilerParams(use_tc_tiling_on_sc=False),
    )
    def gather_kernel(table_hbm, idx_hbm, out_hbm, idx_v, *rows_and_sems):
        rows = rows_and_sems[:NBUF]
        sems = rows_and_sems[NBUF:]
        wid = lax.axis_index("s") * NUM_CORES + lax.axis_index("c")
        base = wid * per_worker
        pltpu.sync_copy(idx_hbm.at[pl.ds(base, per_worker)], idx_v)

        def start(chunk, b):
            pltpu.async_copy(
                table_hbm.at[idx_v.at[pl.ds(chunk * CHUNK, CHUNK)]],
                rows[b],
                sems[b],
            )

        for b in range(NBUF):
            start(b, b)

        @pl.loop(0, num_chunks, step=NBUF)
        def _(c0):
            for b in range(NBUF):
                c = c0 + b
                pltpu.make_async_copy(
                    table_hbm.at[idx_v.at[pl.ds(0, CHUNK)]], rows[b], sems[b]
                ).wait()
                pltpu.sync_copy(
                    rows[b], out_hbm.at[pl.ds(base + c * CHUNK, CHUNK)]
                )
                nxt = c + NBUF

                @pl.when(nxt < num_chunks)
                def _():
                    start(nxt, b)

    return gather_kernel(table, idx_flat)


def _tc_matmul(x2, W, b):
    kdim, out_dim = W.shape
    half = kdim // 2
    num_rows = x2.shape[0] * 128 // half
    xrows_per_block = MM_BLOCK * half // 128
    groups = MM_BLOCK // GROUP
    gwords = GROUP * half

    wbf = W.astype(jnp.bfloat16)
    w_even = jnp.zeros((gwords, GROUP * out_dim), jnp.bfloat16)
    w_odd = jnp.zeros((gwords, GROUP * out_dim), jnp.bfloat16)
    for a in range(GROUP):
        w_even = lax.dynamic_update_slice(
            w_even, wbf[0::2, :], (a * half, a * out_dim)
        )
        w_odd = lax.dynamic_update_slice(
            w_odd, wbf[1::2, :], (a * half, a * out_dim)
        )
    b_tiled = jnp.tile(b, GROUP).reshape(1, GROUP * out_dim)

    def mm_kernel(x_ref, we_ref, wo_ref, b_ref, o_ref):
        xi = pltpu.bitcast(x_ref[...], jnp.int32)
        even = pltpu.bitcast(xi << 16, jnp.float32).astype(jnp.bfloat16)
        odd = pltpu.bitcast(xi & jnp.int32(-65536), jnp.float32).astype(
            jnp.bfloat16
        )
        eg = even.reshape(groups, gwords)
        og = odd.reshape(groups, gwords)
        out = (
            jnp.dot(eg, we_ref[...], preferred_element_type=jnp.float32)
            + jnp.dot(og, wo_ref[...], preferred_element_type=jnp.float32)
            + b_ref[...]
        )
        o_ref[...] = out.reshape(MM_BLOCK, out_dim)

    return pl.pallas_call(
        mm_kernel,
        grid=(num_rows // MM_BLOCK,),
        in_specs=[
            pl.BlockSpec((xrows_per_block, 128), lambda i: (i, 0)),
            pl.BlockSpec(w_even.shape, lambda i: (0, 0)),
            pl.BlockSpec(w_odd.shape, lambda i: (0, 0)),
            pl.BlockSpec(b_tiled.shape, lambda i: (0, 0)),
        ],
        out_specs=pl.BlockSpec((MM_BLOCK, out_dim), lambda i: (i, 0)),
        out_shape=jax.ShapeDtypeStruct((num_rows, out_dim), jnp.float32),
    )(x2, w_even, w_odd, b_tiled)


def kernel(obs, table, W, b):
    batch, seq, obs_dim = obs.shape
    num_indices = batch * seq * obs_dim
    idx_flat = obs.reshape(num_indices)
    table_packed = _pack_bf16_table(table)
    emb = _sc_gather(table_packed, idx_flat)
    emb128 = emb.reshape(num_indices * PACKED // 128, 128)
    out = _tc_matmul(emb128, W, b)
    return out.reshape(batch, seq, -1)


# speedup vs baseline: 2.7826x; 2.7826x over previous
import functools

import jax
import jax.numpy as jnp
from jax import lax
from jax.experimental import pallas as pl
from jax.experimental.pallas import tpu as pltpu
from jax.experimental.pallas import tpu_sc as plsc

EMBED = 16
NUM_CORES = 2
NUM_SUBCORES = 16
NUM_WORKERS = NUM_CORES * NUM_SUBCORES
CHUNK = 128
NBUF = 5
GROUP = 4
MM_BLOCK = 512


def _sc_gather(table, idx_flat):
    num_indices = idx_flat.shape[0]
    per_worker = num_indices // NUM_WORKERS
    num_chunks = per_worker // CHUNK
    assert per_worker % CHUNK == 0 and num_chunks % NBUF == 0

    mesh = plsc.VectorSubcoreMesh(core_axis_name="c", subcore_axis_name="s")

    scratch = [pltpu.VMEM((per_worker,), jnp.int32)]
    scratch += [pltpu.VMEM((CHUNK, EMBED), table.dtype) for _ in range(NBUF)]
    scratch += [pltpu.SemaphoreType.DMA for _ in range(NBUF)]

    @functools.partial(
        pl.kernel,
        mesh=mesh,
        out_type=jax.ShapeDtypeStruct((num_indices, EMBED), table.dtype),
        scratch_types=scratch,
        compiler_params=pltpu.CompilerParams(use_tc_tiling_on_sc=False),
    )
    def gather_kernel(table_hbm, idx_hbm, out_hbm, idx_v, *rows_and_sems):
        rows = rows_and_sems[:NBUF]
        sems = rows_and_sems[NBUF:]
        wid = lax.axis_index("s") * NUM_CORES + lax.axis_index("c")
        base = wid * per_worker
        pltpu.sync_copy(idx_hbm.at[pl.ds(base, per_worker)], idx_v)

        def start(chunk, b):
            pltpu.async_copy(
                table_hbm.at[idx_v.at[pl.ds(chunk * CHUNK, CHUNK)]],
                rows[b],
                sems[b],
            )

        for b in range(NBUF):
            start(b, b)

        @pl.loop(0, num_chunks, step=NBUF)
        def _(c0):
            for b in range(NBUF):
                c = c0 + b
                pltpu.make_async_copy(
                    table_hbm.at[idx_v.at[pl.ds(0, CHUNK)]], rows[b], sems[b]
                ).wait()
                pltpu.sync_copy(
                    rows[b], out_hbm.at[pl.ds(base + c * CHUNK, CHUNK)]
                )
                nxt = c + NBUF

                @pl.when(nxt < num_chunks)
                def _():
                    start(nxt, b)

    return gather_kernel(table, idx_flat)


def _tc_matmul(x2, W, b):
    out_dim = W.shape[1]
    num_rows = x2.shape[0] * 128 // (W.shape[0])
    xrows_per_block = MM_BLOCK * W.shape[0] // 128
    groups = MM_BLOCK // GROUP

    t2 = jnp.zeros((GROUP * W.shape[0] // 128 * 128, GROUP * out_dim), x2.dtype)
    for a in range(GROUP):
        t2 = lax.dynamic_update_slice(t2, W, (a * W.shape[0], a * out_dim))
    b_tiled = jnp.tile(b, GROUP).reshape(1, GROUP * out_dim)

    def mm_kernel(x_ref, t2_ref, b_ref, o_ref):
        xg = x_ref[...].reshape(groups, xrows_per_block * 128 // groups)
        out = (
            jnp.dot(xg, t2_ref[...], preferred_element_type=jnp.float32)
            + b_ref[...]
        )
        o_ref[...] = out.reshape(MM_BLOCK, out_dim)

    return pl.pallas_call(
        mm_kernel,
        grid=(num_rows // MM_BLOCK,),
        in_specs=[
            pl.BlockSpec((xrows_per_block, 128), lambda i: (i, 0)),
            pl.BlockSpec(t2.shape, lambda i: (0, 0)),
            pl.BlockSpec(b_tiled.shape, lambda i: (0, 0)),
        ],
        out_specs=pl.BlockSpec((MM_BLOCK, out_dim), lambda i: (i, 0)),
        out_shape=jax.ShapeDtypeStruct((num_rows, out_dim), jnp.float32),
    )(x2, t2, b_tiled)


def kernel(obs, table, W, b):
    batch, seq, obs_dim = obs.shape
    num_indices = batch * seq * obs_dim
    idx_flat = obs.reshape(num_indices)
    emb = _sc_gather(table, idx_flat)
    emb128 = emb.reshape(num_indices * EMBED // 128, 128)
    out = _tc_matmul(emb128, W, b)
    return out.reshape(batch, seq, -1)


# speedup vs baseline: 3.0166x; 1.0841x over previous
import functools

import jax
import jax.numpy as jnp
from jax import lax
from jax.experimental import pallas as pl
from jax.experimental.pallas import tpu as pltpu
from jax.experimental.pallas import tpu_sc as plsc

EMBED = 16
NUM_CORES = 2
NUM_SUBCORES = 16
NUM_WORKERS = NUM_CORES * NUM_SUBCORES
CHUNK = 128
NBUF = 5
GROUP = 4
MM_BLOCK = 512


def _sc_gather(table, idx_flat):
    num_indices = idx_flat.shape[0]
    per_worker = num_indices // NUM_WORKERS
    num_chunks = per_worker // CHUNK
    assert per_worker % CHUNK == 0 and num_chunks % NBUF == 0

    mesh = plsc.VectorSubcoreMesh(core_axis_name="c", subcore_axis_name="s")

    scratch = [pltpu.VMEM((per_worker,), jnp.int32)]
    scratch += [pltpu.VMEM((CHUNK, EMBED), table.dtype) for _ in range(NBUF)]
    scratch += [pltpu.SemaphoreType.DMA for _ in range(NBUF)]

    @functools.partial(
        pl.kernel,
        mesh=mesh,
        out_type=jax.ShapeDtypeStruct((num_indices, EMBED), table.dtype),
        scratch_types=scratch,
        compiler_params=pltpu.CompilerParams(use_tc_tiling_on_sc=False),
    )
    def gather_kernel(table_hbm, idx_hbm, out_hbm, idx_v, *rows_and_sems):
        rows = rows_and_sems[:NBUF]
        sems = rows_and_sems[NBUF:]
        wid = lax.axis_index("s") * NUM_CORES + lax.axis_index("c")
        base = wid * per_worker
        pltpu.sync_copy(idx_hbm.at[pl.ds(base, per_worker)], idx_v)

        def start(chunk, b):
            pltpu.async_copy(
                table_hbm.at[idx_v.at[pl.ds(chunk * CHUNK, CHUNK)]],
                rows[b],
                sems[b],
            )

        for b in range(NBUF):
            start(b, b)

        @pl.loop(0, num_chunks, step=NBUF)
        def _(c0):
            for b in range(NBUF):
                c = c0 + b
                pltpu.make_async_copy(
                    table_hbm.at[idx_v.at[pl.ds(0, CHUNK)]], rows[b], sems[b]
                ).wait()
                pltpu.sync_copy(
                    rows[b], out_hbm.at[pl.ds(base + c * CHUNK, CHUNK)]
                )
                nxt = c + NBUF

                @pl.when(nxt < num_chunks)
                def _():
                    start(nxt, b)

    return gather_kernel(table, idx_flat)


def _tc_matmul(x2, W, b):
    out_dim = W.shape[1]
    num_rows = x2.shape[0] * 128 // (W.shape[0])
    xrows_per_block = MM_BLOCK * W.shape[0] // 128
    groups = MM_BLOCK // GROUP

    t2 = jnp.zeros((GROUP * W.shape[0] // 128 * 128, GROUP * out_dim), x2.dtype)
    for a in range(GROUP):
        t2 = lax.dynamic_update_slice(t2, W, (a * W.shape[0], a * out_dim))
    b_tiled = jnp.tile(b, GROUP).reshape(1, GROUP * out_dim)

    def mm_kernel(x_ref, t2_ref, b_ref, o_ref):
        xg = x_ref[...].reshape(groups, xrows_per_block * 128 // groups)
        out = (
            jnp.dot(xg, t2_ref[...], preferred_element_type=jnp.float32)
            + b_ref[...]
        )
        o_ref[...] = out.reshape(MM_BLOCK, out_dim)

    return pl.pallas_call(
        mm_kernel,
        grid=(num_rows // MM_BLOCK,),
        in_specs=[
            pl.BlockSpec((xrows_per_block, 128), lambda i: (i, 0)),
            pl.BlockSpec(t2.shape, lambda i: (0, 0)),
            pl.BlockSpec(b_tiled.shape, lambda i: (0, 0)),
        ],
        out_specs=pl.BlockSpec((MM_BLOCK, out_dim), lambda i: (i, 0)),
        out_shape=jax.ShapeDtypeStruct((num_rows, out_dim), jnp.float32),
    )(x2, t2, b_tiled)


def kernel(obs, table, W, b):
    batch, seq, obs_dim = obs.shape
    num_indices = batch * seq * obs_dim
    idx_flat = obs.transpose(1, 0, 2).reshape(num_indices)
    emb = _sc_gather(table, idx_flat)
    emb128 = emb.reshape(num_indices * EMBED // 128, 128)
    out = _tc_matmul(emb128, W, b)
    return out.reshape(seq, batch, -1).transpose(1, 0, 2)


# speedup vs baseline: 3.3308x; 1.1042x over previous
import functools

import jax
import jax.numpy as jnp
from jax import lax
from jax.experimental import pallas as pl
from jax.experimental.pallas import tpu as pltpu
from jax.experimental.pallas import tpu_sc as plsc

EMBED = 16
NUM_CORES = 2
NUM_SUBCORES = 16
NUM_WORKERS = NUM_CORES * NUM_SUBCORES
CHUNK = 128
NBUF = 5
GROUP = 4
MM_BLOCK = 512


def _sc_detile(table_t):
    vocab = table_t.shape[1]
    full_tiles = vocab // 128
    out_rows = vocab * EMBED // 128

    mesh = plsc.VectorSubcoreMesh(core_axis_name="c", subcore_axis_name="s")

    scratch = [pltpu.VMEM((EMBED, 128), jnp.float32) for _ in range(2)]
    scratch += [pltpu.VMEM((EMBED, 128), jnp.float32) for _ in range(2)]
    scratch += [pltpu.SemaphoreType.DMA for _ in range(4)]

    @functools.partial(
        pl.kernel,
        mesh=mesh,
        out_type=jax.ShapeDtypeStruct((out_rows, 128), jnp.float32),
        scratch_types=scratch,
        compiler_params=pltpu.CompilerParams(
            use_tc_tiling_on_sc=True, needs_layout_passes=False
        ),
    )
    def detile_kernel(t_hbm, out_hbm, in0, in1, o0, o1, si0, si1, so0, so1):
        wid = lax.axis_index("s") * NUM_CORES + lax.axis_index("c")
        ins, outs = (in0, in1), (o0, o1)
        isems, osems = (si0, si1), (so0, so1)
        e_iota = lax.broadcasted_iota(jnp.int32, (16,), 0)
        stride = 2 * NUM_WORKERS

        def start_in(j, p):
            pltpu.async_copy(
                t_hbm.at[:, pl.ds(j * 128, 128)], ins[p], isems[p]
            )

        def transpose_tile(p, cols):
            @pl.loop(0, cols // 8)
            def _(r):
                for q in range(8):
                    l = 8 * r + q
                    vals = plsc.load_gather(
                        ins[p], [e_iota, jnp.full((16,), l, jnp.int32)]
                    )
                    outs[p][r, pl.ds(q * 16, 16)] = vals

        for p in range(2):
            @pl.when(wid + p * NUM_WORKERS < full_tiles)
            def _():
                start_in(wid + p * NUM_WORKERS, p)

        @pl.loop(wid, full_tiles, step=stride)
        def _(j0):
            for p in range(2):
                j = j0 + p * NUM_WORKERS

                @pl.when(j < full_tiles)
                def _():
                    pltpu.make_async_copy(
                        t_hbm.at[:, pl.ds(0, 128)], ins[p], isems[p]
                    ).wait()

                    @pl.when(j >= wid + stride)
                    def _():
                        pltpu.make_async_copy(
                            outs[p], out_hbm.at[pl.ds(0, EMBED)], osems[p]
                        ).wait()

                    transpose_tile(p, 128)

                    @pl.when(j + stride < full_tiles)
                    def _():
                        start_in(j + stride, p)

                    pltpu.async_copy(
                        outs[p],
                        out_hbm.at[pl.ds(j * EMBED, EMBED)],
                        osems[p],
                    )

        for p in range(2):
            @pl.when(wid + p * NUM_WORKERS < full_tiles)
            def _():
                pltpu.make_async_copy(
                    outs[p], out_hbm.at[pl.ds(0, EMBED)], osems[p]
                ).wait()

    return detile_kernel(table_t)


def _sc_gather(table, idx_flat):
    num_indices = idx_flat.shape[0]
    per_worker = num_indices // NUM_WORKERS
    num_chunks = per_worker // CHUNK
    assert per_worker % CHUNK == 0 and num_chunks % NBUF == 0

    mesh = plsc.VectorSubcoreMesh(core_axis_name="c", subcore_axis_name="s")

    scratch = [pltpu.VMEM((per_worker,), jnp.int32)]
    scratch += [pltpu.VMEM((CHUNK, EMBED), table.dtype) for _ in range(NBUF)]
    scratch += [pltpu.SemaphoreType.DMA for _ in range(NBUF)]

    @functools.partial(
        pl.kernel,
        mesh=mesh,
        out_type=jax.ShapeDtypeStruct((num_indices, EMBED), table.dtype),
        scratch_types=scratch,
        compiler_params=pltpu.CompilerParams(use_tc_tiling_on_sc=False),
    )
    def gather_kernel(table_hbm, idx_hbm, out_hbm, idx_v, *rows_and_sems):
        rows = rows_and_sems[:NBUF]
        sems = rows_and_sems[NBUF:]
        wid = lax.axis_index("s") * NUM_CORES + lax.axis_index("c")
        base = wid * per_worker
        pltpu.sync_copy(idx_hbm.at[pl.ds(base, per_worker)], idx_v)

        def start(chunk, b):
            pltpu.async_copy(
                table_hbm.at[idx_v.at[pl.ds(chunk * CHUNK, CHUNK)]],
                rows[b],
                sems[b],
            )

        for b in range(NBUF):
            start(b, b)

        @pl.loop(0, num_chunks, step=NBUF)
        def _(c0):
            for b in range(NBUF):
                c = c0 + b
                pltpu.make_async_copy(
                    table_hbm.at[idx_v.at[pl.ds(0, CHUNK)]], rows[b], sems[b]
                ).wait()
                pltpu.sync_copy(
                    rows[b], out_hbm.at[pl.ds(base + c * CHUNK, CHUNK)]
                )
                nxt = c + NBUF

                @pl.when(nxt < num_chunks)
                def _():
                    start(nxt, b)

    return gather_kernel(table, idx_flat)


def _tc_matmul(x2, W, b):
    out_dim = W.shape[1]
    num_rows = x2.shape[0] * 128 // (W.shape[0])
    xrows_per_block = MM_BLOCK * W.shape[0] // 128
    groups = MM_BLOCK // GROUP

    t2 = jnp.zeros((GROUP * W.shape[0] // 128 * 128, GROUP * out_dim), x2.dtype)
    for a in range(GROUP):
        t2 = lax.dynamic_update_slice(t2, W, (a * W.shape[0], a * out_dim))
    b_tiled = jnp.tile(b, GROUP).reshape(1, GROUP * out_dim)

    def mm_kernel(x_ref, t2_ref, b_ref, o_ref):
        xg = x_ref[...].reshape(groups, xrows_per_block * 128 // groups)
        out = (
            jnp.dot(xg, t2_ref[...], preferred_element_type=jnp.float32)
            + b_ref[...]
        )
        o_ref[...] = out.reshape(MM_BLOCK, out_dim)

    return pl.pallas_call(
        mm_kernel,
        grid=(num_rows // MM_BLOCK,),
        in_specs=[
            pl.BlockSpec((xrows_per_block, 128), lambda i: (i, 0)),
            pl.BlockSpec(t2.shape, lambda i: (0, 0)),
            pl.BlockSpec(b_tiled.shape, lambda i: (0, 0)),
        ],
        out_specs=pl.BlockSpec((MM_BLOCK, out_dim), lambda i: (i, 0)),
        out_shape=jax.ShapeDtypeStruct((num_rows, out_dim), jnp.float32),
    )(x2, t2, b_tiled)


def kernel(obs, table, W, b):
    batch, seq, obs_dim = obs.shape
    num_indices = batch * seq * obs_dim
    idx_flat = obs.transpose(1, 0, 2).reshape(num_indices)
    t128 = _sc_detile(table.T)
    vocab = table.shape[0]
    full_tiles = vocab // 128
    if vocab % 128:
        tail = table[full_tiles * 128:, :].reshape(-1, 128)
        t128 = lax.dynamic_update_slice(
            t128, tail, (full_tiles * EMBED, 0)
        )
    table_lin = t128.reshape(table.shape)
    emb = _sc_gather(table_lin, idx_flat)
    emb128 = emb.reshape(num_indices * EMBED // 128, 128)
    out = _tc_matmul(emb128, W, b)
    return out.reshape(seq, batch, -1).transpose(1, 0, 2)
